# Initial kernel scaffold; baseline (speedup 1.0000x reference)
import jax
import jax.numpy as jnp
from jax.experimental import pallas as pl


def kernel(idx, tok_table, pos_table, ln_gamma, ln_beta, W_lm):
    raise NotImplementedError("write your pallas kernel here")


# speedup vs baseline: 1.3339x; 1.3339x over previous
import functools

import jax
import jax.numpy as jnp
from jax import lax
from jax.experimental import pallas as pl
from jax.experimental.pallas import tpu as pltpu
from jax.experimental.pallas import tpu_sc as plsc

_VOCAB = 50257
_D = 2048
_T = 2048

_NC = 2
_NS = 16
_NW = _NC * _NS
_ROWS_PER_W = _T // _NW
_CHUNK = 32
_NCHUNK = _ROWS_PER_W // _CHUNK


def _sc_gather_body(table_hbm, idx_hbm, out_hbm, idx_v, rows_v, sem):
    wid = lax.axis_index("s") * _NC + lax.axis_index("c")
    base = wid * _ROWS_PER_W
    for c in range(_NCHUNK):
        off = base + c * _CHUNK
        pltpu.sync_copy(idx_hbm.at[pl.ds(off, _CHUNK)], idx_v.at[c])
        pltpu.async_copy(table_hbm.at[idx_v.at[c]], rows_v, sem).wait()
        pltpu.sync_copy(rows_v, out_hbm.at[pl.ds(off, _CHUNK)])


@jax.jit
def _sc_gather(table, idx_flat):
    mesh = plsc.VectorSubcoreMesh(core_axis_name="c", subcore_axis_name="s")
    f = pl.kernel(
        _sc_gather_body,
        out_type=jax.ShapeDtypeStruct((_T, _D), jnp.float32),
        mesh=mesh,
        scratch_types=[
            pltpu.VMEM((_NCHUNK, _CHUNK), jnp.int32),
            pltpu.VMEM((_CHUNK, _D), jnp.float32),
            pltpu.SemaphoreType.DMA,
        ],
    )
    return f(table, idx_flat)


def _ln_body(tok_ref, pos_ref, g_ref, b_ref, o_ref):
    x = tok_ref[...] + pos_ref[...]
    m = jnp.mean(x, axis=-1, keepdims=True)
    xc = x - m
    v = jnp.mean(xc * xc, axis=-1, keepdims=True)
    y = xc * lax.rsqrt(v + 1e-5) * g_ref[...] + b_ref[...]
    o_ref[...] = y.astype(jnp.bfloat16)


_LN_BT = 256


def _ln(tok_emb, pos, gamma, beta):
    return pl.pallas_call(
        _ln_body,
        grid=(_T // _LN_BT,),
        in_specs=[
            pl.BlockSpec((_LN_BT, _D), lambda t: (t, 0)),
            pl.BlockSpec((_LN_BT, _D), lambda t: (t, 0)),
            pl.BlockSpec((1, _D), lambda t: (0, 0)),
            pl.BlockSpec((1, _D), lambda t: (0, 0)),
        ],
        out_specs=pl.BlockSpec((_LN_BT, _D), lambda t: (t, 0)),
        out_shape=jax.ShapeDtypeStruct((_T, _D), jnp.bfloat16),
    )(tok_emb, pos, gamma, beta)


def _mm_body(x_ref, w_ref, o_ref):
    w = w_ref[...].astype(jnp.bfloat16)
    o_ref[...] = lax.dot_general(
        x_ref[...], w, (((1,), (1,)), ((), ())),
        preferred_element_type=jnp.float32)


_MM_BV = 512


def _mm(x_bf16, w):
    nv = pl.cdiv(_VOCAB, _MM_BV)
    return pl.pallas_call(
        _mm_body,
        grid=(nv,),
        in_specs=[
            pl.BlockSpec((_T, _D), lambda v: (0, 0)),
            pl.BlockSpec((_MM_BV, _D), lambda v: (v, 0)),
        ],
        out_specs=pl.BlockSpec((_T, _MM_BV), lambda v: (0, v)),
        out_shape=jax.ShapeDtypeStruct((_T, _VOCAB), jnp.float32),
    )(x_bf16, w)


def kernel(idx, tok_table, pos_table, ln_gamma, ln_beta, W_lm):
    idx_flat = idx.reshape(_T).astype(jnp.int32)
    tok_emb = _sc_gather(tok_table, idx_flat)
    x = _ln(tok_emb, pos_table,
            ln_gamma.reshape(1, _D), ln_beta.reshape(1, _D))
    logits = _mm(x, W_lm)
    return logits[None]


# speedup vs baseline: 1.3554x; 1.0161x over previous
import functools

import jax
import jax.numpy as jnp
from jax import lax
from jax.experimental import pallas as pl
from jax.experimental.pallas import tpu as pltpu
from jax.experimental.pallas import tpu_sc as plsc

_VOCAB = 50257
_D = 2048
_T = 2048

_NC = 2
_NS = 16
_NW = _NC * _NS
_ROWS_PER_W = _T // _NW
_CHUNK = 32
_NCHUNK = _ROWS_PER_W // _CHUNK


def _sc_gather_body(table_hbm, idx_hbm, out_hbm, idx_v, rows_v, sem):
    wid = lax.axis_index("s") * _NC + lax.axis_index("c")
    base = wid * _ROWS_PER_W
    for c in range(_NCHUNK):
        off = base + c * _CHUNK
        pltpu.sync_copy(idx_hbm.at[pl.ds(off, _CHUNK)], idx_v.at[c])
        pltpu.async_copy(table_hbm.at[idx_v.at[c]], rows_v, sem).wait()
        pltpu.sync_copy(rows_v, out_hbm.at[pl.ds(off, _CHUNK)])


@jax.jit
def _sc_gather(table, idx_flat):
    mesh = plsc.VectorSubcoreMesh(core_axis_name="c", subcore_axis_name="s")
    f = pl.kernel(
        _sc_gather_body,
        out_type=jax.ShapeDtypeStruct((_T, _D), jnp.float32),
        mesh=mesh,
        scratch_types=[
            pltpu.VMEM((_NCHUNK, _CHUNK), jnp.int32),
            pltpu.VMEM((_CHUNK, _D), jnp.float32),
            pltpu.SemaphoreType.DMA,
        ],
    )
    return f(table, idx_flat)


def _ln_body(tok_ref, pos_ref, g_ref, b_ref, o_ref):
    x = tok_ref[...] + pos_ref[...]
    m = jnp.mean(x, axis=-1, keepdims=True)
    xc = x - m
    v = jnp.mean(xc * xc, axis=-1, keepdims=True)
    y = xc * lax.rsqrt(v + 1e-5) * g_ref[...] + b_ref[...]
    o_ref[...] = y.astype(jnp.bfloat16)


_LN_BT = 256


def _ln(tok_emb, pos, gamma, beta):
    return pl.pallas_call(
        _ln_body,
        grid=(_T // _LN_BT,),
        in_specs=[
            pl.BlockSpec((_LN_BT, _D), lambda t: (t, 0)),
            pl.BlockSpec((_LN_BT, _D), lambda t: (t, 0)),
            pl.BlockSpec((1, _D), lambda t: (0, 0)),
            pl.BlockSpec((1, _D), lambda t: (0, 0)),
        ],
        out_specs=pl.BlockSpec((_LN_BT, _D), lambda t: (t, 0)),
        out_shape=jax.ShapeDtypeStruct((_T, _D), jnp.bfloat16),
    )(tok_emb, pos, gamma, beta)


def _mm_body(x_ref, w_ref, o_ref):
    w = w_ref[...].astype(jnp.bfloat16)
    o_ref[...] = lax.dot_general(
        x_ref[...], w, (((1,), (1,)), ((), ())),
        preferred_element_type=jnp.float32)


_MM_BV = 1024


def _mm(x_bf16, w):
    nv = pl.cdiv(_VOCAB, _MM_BV)
    return pl.pallas_call(
        _mm_body,
        grid=(nv,),
        in_specs=[
            pl.BlockSpec((_T, _D), lambda v: (0, 0)),
            pl.BlockSpec((_MM_BV, _D), lambda v: (v, 0)),
        ],
        out_specs=pl.BlockSpec((_T, _MM_BV), lambda v: (0, v)),
        out_shape=jax.ShapeDtypeStruct((_T, _VOCAB), jnp.float32),
    )(x_bf16, w)


def kernel(idx, tok_table, pos_table, ln_gamma, ln_beta, W_lm):
    idx_flat = idx.reshape(_T).astype(jnp.int32)
    tok_emb = _sc_gather(tok_table, idx_flat)
    x = _ln(tok_emb, pos_table,
            ln_gamma.reshape(1, _D), ln_beta.reshape(1, _D))
    logits = _mm(x, W_lm)
    return logits[None]
